# Initial kernel scaffold; baseline (speedup 1.0000x reference)
import jax
import jax.numpy as jnp
from jax.experimental import pallas as pl


def kernel(prediction, target):
    raise NotImplementedError("write your pallas kernel here")


# speedup vs baseline: 1.3391x; 1.3391x over previous
import functools

import jax
import jax.numpy as jnp
from jax import lax
from jax.experimental import pallas as pl
from jax.experimental.pallas import tpu as pltpu
from jax.experimental.pallas import tpu_sc as plsc

B = 16384
C = 1000
CPAD = 1024
CROWS = CPAD // 16
ROW_BLOCK = 512
NB = B // ROW_BLOCK
NS = 16
CHUNK = B // NS
L = 16


def _entropy_body(x_ref, out_ref):
    x = x_ref[...]
    m = jnp.max(x, axis=1, keepdims=True)
    e = jnp.exp(x - m)
    s = jnp.sum(e, axis=1)
    u = jnp.sum(e * x, axis=1)
    ent = m[:, 0] + jnp.log(s) - u / s
    out_ref[...] = ent.reshape(1, 1, ROW_BLOCK)


def _rowwise_entropy(prediction):
    return pl.pallas_call(
        _entropy_body,
        grid=(NB,),
        in_specs=[pl.BlockSpec((ROW_BLOCK, C), lambda i: (i, 0))],
        out_specs=pl.BlockSpec((1, 1, ROW_BLOCK), lambda i: (i, 0, 0)),
        out_shape=jax.ShapeDtypeStruct((NB, 1, ROW_BLOCK), jnp.float32),
    )(prediction)


def _hist_body(ent_hbm, tgt_hbm, out_hbm,
               tgt_v, val_v, hist_v, part_v, out_v, shared):
    c = lax.axis_index("c")
    s = lax.axis_index("s")
    base = s * CHUNK

    zeros16 = jnp.zeros((L,), jnp.float32)

    def zero_body(i, _):
        hist_v[pl.ds(i * L, L)] = zeros16
        return 0
    lax.fori_loop(0, CPAD // L, zero_body, 0)

    pltpu.sync_copy(tgt_hbm.at[pl.ds(base, CHUNK)], tgt_v)

    @pl.when(c == 0)
    def _():
        pltpu.sync_copy(ent_hbm.at[pl.ds(base, CHUNK)], val_v)

    @pl.when(c != 0)
    def _():
        ones16 = jnp.ones((L,), jnp.float32)
        def ones_body(i, _):
            val_v[pl.ds(i * L, L)] = ones16
            return 0
        lax.fori_loop(0, CHUNK // L, ones_body, 0)

    def scat_body(j, _):
        idx = tgt_v[pl.ds(j * L, L)]
        val = val_v[pl.ds(j * L, L)]
        plsc.addupdate_scatter(hist_v, [idx], val)
        return 0
    lax.fori_loop(0, CHUNK // L, scat_body, 0)

    pltpu.sync_copy(hist_v, shared.at[s])
    plsc.subcore_barrier()
    pltpu.sync_copy(shared, part_v)

    span = CPAD // NS
    for k in range(span // L):
        acc = zeros16
        for r in range(NS):
            acc = acc + part_v[r, pl.ds(s * span + k * L, L)]
        out_v[pl.ds(k * L, L)] = acc

    pltpu.sync_copy(out_v, out_hbm.at[pl.ds(c * CPAD + s * span, span)])


@functools.cache
def _hist_call():
    return pl.kernel(
        _hist_body,
        out_type=jax.ShapeDtypeStruct((2 * CPAD,), jnp.float32),
        mesh=plsc.VectorSubcoreMesh(core_axis_name="c", subcore_axis_name="s"),
        compiler_params=pltpu.CompilerParams(needs_layout_passes=False),
        scratch_types=[
            pltpu.VMEM((CHUNK,), jnp.int32),
            pltpu.VMEM((CHUNK,), jnp.float32),
            pltpu.VMEM((CPAD,), jnp.float32),
            pltpu.VMEM((NS, CPAD), jnp.float32),
            pltpu.VMEM((CPAD // NS,), jnp.float32),
            pltpu.VMEM_SHARED((NS, CPAD), jnp.float32),
        ],
    )


def kernel(prediction, target):
    ent = _rowwise_entropy(prediction).reshape(B)
    tgt = target.astype(jnp.int32)
    out = _hist_call()(ent, tgt)
    return out[:C], out[CPAD:CPAD + C]


# speedup vs baseline: 1.4662x; 1.0949x over previous
import functools

import jax
import jax.numpy as jnp
from jax import lax
from jax.experimental import pallas as pl
from jax.experimental.pallas import tpu as pltpu
from jax.experimental.pallas import tpu_sc as plsc

B = 16384
C = 1000
CPAD = 1024
CROWS = CPAD // 16
ROW_BLOCK = 2048
NB = B // ROW_BLOCK
NS = 16
CHUNK = B // NS
L = 16


def _entropy_body(x_ref, out_ref):
    x = x_ref[...]
    m = jnp.max(x, axis=1, keepdims=True)
    e = jnp.exp(x - m)
    s = jnp.sum(e, axis=1)
    u = jnp.sum(e * x, axis=1)
    ent = m[:, 0] + jnp.log(s) - u / s
    out_ref[...] = ent.reshape(1, 1, ROW_BLOCK)


def _rowwise_entropy(prediction):
    return pl.pallas_call(
        _entropy_body,
        grid=(NB,),
        in_specs=[pl.BlockSpec((ROW_BLOCK, C), lambda i: (i, 0))],
        out_specs=pl.BlockSpec((1, 1, ROW_BLOCK), lambda i: (i, 0, 0)),
        out_shape=jax.ShapeDtypeStruct((NB, 1, ROW_BLOCK), jnp.float32),
    )(prediction)


def _hist_body(ent_hbm, tgt_hbm, out_hbm,
               tgt_v, val_v, hist_v, part_v, out_v, shared):
    c = lax.axis_index("c")
    s = lax.axis_index("s")
    base = s * CHUNK

    zeros16 = jnp.zeros((L,), jnp.float32)

    def zero_body(i, _):
        hist_v[pl.ds(i * L, L)] = zeros16
        return 0
    lax.fori_loop(0, CPAD // L, zero_body, 0)

    pltpu.sync_copy(tgt_hbm.at[pl.ds(base, CHUNK)], tgt_v)

    @pl.when(c == 0)
    def _():
        pltpu.sync_copy(ent_hbm.at[pl.ds(base, CHUNK)], val_v)

    @pl.when(c != 0)
    def _():
        ones16 = jnp.ones((L,), jnp.float32)
        def ones_body(i, _):
            val_v[pl.ds(i * L, L)] = ones16
            return 0
        lax.fori_loop(0, CHUNK // L, ones_body, 0)

    def scat_body(j, _):
        idx = tgt_v[pl.ds(j * L, L)]
        val = val_v[pl.ds(j * L, L)]
        plsc.addupdate_scatter(hist_v, [idx], val)
        return 0
    lax.fori_loop(0, CHUNK // L, scat_body, 0)

    pltpu.sync_copy(hist_v, shared.at[s])
    plsc.subcore_barrier()
    pltpu.sync_copy(shared, part_v)

    span = CPAD // NS
    for k in range(span // L):
        acc = zeros16
        for r in range(NS):
            acc = acc + part_v[r, pl.ds(s * span + k * L, L)]
        out_v[pl.ds(k * L, L)] = acc

    pltpu.sync_copy(out_v, out_hbm.at[pl.ds(c * CPAD + s * span, span)])


@functools.cache
def _hist_call():
    return pl.kernel(
        _hist_body,
        out_type=jax.ShapeDtypeStruct((2 * CPAD,), jnp.float32),
        mesh=plsc.VectorSubcoreMesh(core_axis_name="c", subcore_axis_name="s"),
        compiler_params=pltpu.CompilerParams(needs_layout_passes=False),
        scratch_types=[
            pltpu.VMEM((CHUNK,), jnp.int32),
            pltpu.VMEM((CHUNK,), jnp.float32),
            pltpu.VMEM((CPAD,), jnp.float32),
            pltpu.VMEM((NS, CPAD), jnp.float32),
            pltpu.VMEM((CPAD // NS,), jnp.float32),
            pltpu.VMEM_SHARED((NS, CPAD), jnp.float32),
        ],
    )


def kernel(prediction, target):
    ent = _rowwise_entropy(prediction).reshape(B)
    tgt = target.astype(jnp.int32)
    out = _hist_call()(ent, tgt)
    return out[:C], out[CPAD:CPAD + C]
